# baseline (speedup 1.0000x reference)
import functools

import jax
import jax.numpy as jnp
import numpy as np
from jax import lax
from jax.experimental import pallas as pl
from jax.experimental.pallas import tpu as pltpu

_BLK = 512

_ARB = pltpu.CompilerParams(dimension_semantics=("arbitrary",))


def _attn_body(adj0_ref, adj1_ref, aw_ref, z_ref):
    a0 = adj0_ref[...]
    a1 = adj1_ref[...]
    w = aw_ref[...]
    z_ref[:, 0:1] = jnp.sum(a0 * w, axis=1, keepdims=True)
    z_ref[:, 1:2] = jnp.sum(a1 * w, axis=1, keepdims=True)


def _mix_l1_body(adj0_ref, adj1_ref, zall_ref, ab_ref, x_ref, w1_ref, b1_ref,
                 adj_ref, h1_ref, nz0_ref, nz1_ref, s1_ref):
    i = pl.program_id(0)

    @pl.when(i == 0)
    def _():
        one = jnp.ones((1, 1), jnp.float32)
        dn_t = (((1,), (1,)), ((), ()))
        z1 = lax.dot_general(one, zall_ref[:, 0:1], dn_t,
                             preferred_element_type=jnp.float32)
        z2 = lax.dot_general(one, zall_ref[:, 1:2], dn_t,
                             preferred_element_type=jnp.float32)
        z1 = z1 + ab_ref[...]
        z2 = z2 + ab_ref[...]
        m = jnp.maximum(z1, z2)
        e1 = jnp.exp(z1 - m)
        e2 = jnp.exp(z2 - m)
        den = e1 + e2
        nz0_ref[...] = e1 / den
        nz1_ref[...] = e2 / den
        s1_ref[...] = jnp.dot(x_ref[...], w1_ref[...],
                              preferred_element_type=jnp.float32)

    am = nz0_ref[...] * adj0_ref[...] + nz1_ref[...] * adj1_ref[...]
    adj_ref[...] = am
    h1_ref[...] = jnp.maximum(
        jnp.dot(am, s1_ref[...], preferred_element_type=jnp.float32)
        + b1_ref[...], 0.0)


def _layer_body(adj_ref, hin_ref, w_ref, b_ref, hout_ref, s_ref):
    i = pl.program_id(0)

    @pl.when(i == 0)
    def _():
        s_ref[...] = jnp.dot(hin_ref[...], w_ref[...],
                             preferred_element_type=jnp.float32)

    hout_ref[...] = jnp.maximum(
        jnp.dot(adj_ref[...], s_ref[...], preferred_element_type=jnp.float32)
        + b_ref[...], 0.0)


def _out_body(adj_ref, hin_ref, w_ref, b_ref, out_ref, s_ref):
    i = pl.program_id(0)

    @pl.when(i == 0)
    def _():
        s_ref[...] = jnp.dot(hin_ref[...], w_ref[...],
                             preferred_element_type=jnp.float32)

    zz = jnp.dot(adj_ref[...], s_ref[...],
                 preferred_element_type=jnp.float32) + b_ref[...]
    m = jnp.max(zz, axis=1, keepdims=True)
    e = jnp.exp(zz - m)
    out_ref[...] = e / jnp.sum(e, axis=1, keepdims=True)


def _row_map(i):
    return (i, 0)


def _const_map(i):
    return (0, 0)


def _shard_fn(adj0_l, adj1_l, x, aw, ab, w1, b1, wm, bm, w2, b2, axis):
    nl, n = adj0_l.shape
    f = x.shape[1]
    h = w1.shape[1]
    c = w2.shape[1]
    nblk = nl // _BLK
    f32 = jnp.float32

    adj_spec = pl.BlockSpec((_BLK, n), _row_map)

    z_l = pl.pallas_call(
        _attn_body,
        grid=(nblk,),
        in_specs=[adj_spec, adj_spec, pl.BlockSpec((1, n), _const_map)],
        out_specs=pl.BlockSpec((_BLK, 2), _row_map),
        out_shape=jax.ShapeDtypeStruct((nl, 2), f32),
        compiler_params=_ARB,
    )(adj0_l, adj1_l, aw)

    z_all = lax.all_gather(z_l, axis, axis=0, tiled=True)

    adj_l, h1_l = pl.pallas_call(
        _mix_l1_body,
        grid=(nblk,),
        in_specs=[
            adj_spec,
            adj_spec,
            pl.BlockSpec((n, 2), _const_map),
            pl.BlockSpec((1, 1), _const_map),
            pl.BlockSpec((n, f), _const_map),
            pl.BlockSpec((f, h), _const_map),
            pl.BlockSpec((1, h), _const_map),
        ],
        out_specs=[pl.BlockSpec((_BLK, n), _row_map),
                   pl.BlockSpec((_BLK, h), _row_map)],
        out_shape=[jax.ShapeDtypeStruct((nl, n), f32),
                   jax.ShapeDtypeStruct((nl, h), f32)],
        scratch_shapes=[
            pltpu.VMEM((1, n), f32),
            pltpu.VMEM((1, n), f32),
            pltpu.VMEM((n, h), f32),
        ],
        compiler_params=_ARB,
    )(adj0_l, adj1_l, z_all, ab, x, w1, b1)

    h1 = lax.all_gather(h1_l, axis, axis=0, tiled=True)

    h2_l = pl.pallas_call(
        _layer_body,
        grid=(nblk,),
        in_specs=[
            adj_spec,
            pl.BlockSpec((n, h), _const_map),
            pl.BlockSpec((h, h), _const_map),
            pl.BlockSpec((1, h), _const_map),
        ],
        out_specs=pl.BlockSpec((_BLK, h), _row_map),
        out_shape=jax.ShapeDtypeStruct((nl, h), f32),
        scratch_shapes=[pltpu.VMEM((n, h), f32)],
        compiler_params=_ARB,
    )(adj_l, h1, wm, bm)

    h2 = lax.all_gather(h2_l, axis, axis=0, tiled=True)

    out_l = pl.pallas_call(
        _out_body,
        grid=(nblk,),
        in_specs=[
            adj_spec,
            pl.BlockSpec((n, h), _const_map),
            pl.BlockSpec((h, c), _const_map),
            pl.BlockSpec((1, c), _const_map),
        ],
        out_specs=pl.BlockSpec((_BLK, c), _row_map),
        out_shape=jax.ShapeDtypeStruct((nl, c), f32),
        scratch_shapes=[pltpu.VMEM((n, c), f32)],
        compiler_params=_ARB,
    )(adj_l, h2, w2, b2)

    return out_l


def kernel(adj0, adj1, x, adj_origin, atten_w, atten_b, gcn1_w, gcn1_b,
           gcn_w, gcn_b, gcn2_w, gcn2_b):
    del adj_origin
    n = adj0.shape[0]
    h = gcn1_w.shape[1]
    c = gcn2_w.shape[1]

    ab = atten_b.reshape(1, 1).astype(jnp.float32)
    b1 = gcn1_b.reshape(1, h)
    bm = gcn_b.reshape(1, h)
    b2 = gcn2_b.reshape(1, c)

    devs = jax.devices()
    nd = 2 if len(devs) >= 2 and n % (2 * _BLK) == 0 else 1
    mesh = jax.sharding.Mesh(np.array(devs[:nd]), ("r",))
    ps = jax.sharding.PartitionSpec
    row = ps("r", None)
    rep = ps(None, None)

    fn = jax.shard_map(
        functools.partial(_shard_fn, axis="r"),
        mesh=mesh,
        in_specs=(row, row, rep, rep, rep, rep, rep, rep, rep, rep, rep),
        out_specs=row,
        check_vma=False,
    )
    return fn(adj0, adj1, x, atten_w, ab, gcn1_w, b1, gcn_w, bm, gcn2_w, b2)
